# Initial kernel scaffold; baseline (speedup 1.0000x reference)
import functools

import jax
import jax.numpy as jnp
from jax import lax
from jax.experimental import pallas as pl
from jax.experimental.pallas import tpu as pltpu
from jax.experimental.pallas import tpu_sc as plsc

N = 10000
NPAD = 10240
E = 320000
DIN = 128
DH = 64
G = 64
NC = 2
NS = 16
NW = NC * NS
EPW = E // NW
K = 80
NCHUNK = EPW // K
RPT = NPAD // NS

_MESH = dict(core_axis_name="c", subcore_axis_name="s")


@functools.partial(
    pl.kernel,
    out_type=jax.ShapeDtypeStruct((NC * NPAD,), jnp.float32),
    mesh=plsc.VectorSubcoreMesh(**_MESH),
    scratch_types=[
        pltpu.VMEM((K,), jnp.int32),
        pltpu.VMEM((K,), jnp.float32),
        pltpu.VMEM((RPT,), jnp.float32),
        pltpu.VMEM_SHARED((NPAD,), jnp.float32),
    ],
)
def _deg(dst_hbm, out_hbm, idx_v, ones_v, buf_v, acc):
    c = lax.axis_index("c")
    s = lax.axis_index("s")
    wid = s * NC + c

    def fill_ones(i, carry):
        ones_v[pl.ds(i * 16, 16)] = jnp.ones((16,), jnp.float32)
        return carry
    lax.fori_loop(0, K // 16, fill_ones, 0)

    def fill_zeros(i, carry):
        buf_v[pl.ds(i * 16, 16)] = jnp.zeros((16,), jnp.float32)
        return carry
    lax.fori_loop(0, RPT // 16, fill_zeros, 0)

    pltpu.sync_copy(buf_v, acc.at[pl.ds(s * RPT, RPT)])
    plsc.subcore_barrier()

    def chunk(i, carry):
        base = wid * EPW + i * K
        pltpu.sync_copy(dst_hbm.at[pl.ds(base, K)], idx_v)
        pltpu.sync_copy(ones_v, acc.at[idx_v], add=True)
        return carry
    lax.fori_loop(0, NCHUNK, chunk, 0)

    plsc.subcore_barrier()
    pltpu.sync_copy(acc.at[pl.ds(s * RPT, RPT)], buf_v)
    pltpu.sync_copy(buf_v, out_hbm.at[pl.ds(c * NPAD + s * RPT, RPT)])


@functools.partial(
    pl.kernel,
    out_type=jax.ShapeDtypeStruct((NC * NPAD, DH), jnp.float32),
    mesh=plsc.VectorSubcoreMesh(**_MESH),
    scratch_types=[
        pltpu.VMEM((K,), jnp.int32),
        pltpu.VMEM((K,), jnp.int32),
        pltpu.VMEM((K, DH), jnp.float32),
        pltpu.VMEM((RPT, DH), jnp.float32),
        pltpu.VMEM_SHARED((NPAD, DH), jnp.float32),
        pltpu.SemaphoreType.DMA,
    ],
)
def _agg(u_hbm, src_hbm, dst_hbm, out_hbm, sidx, didx, rows, buf, acc, sem):
    c = lax.axis_index("c")
    s = lax.axis_index("s")
    wid = s * NC + c

    def fz(i, carry):
        for j in range(DH // 16):
            rows[i, pl.ds(j * 16, 16)] = jnp.zeros((16,), jnp.float32)
        return carry
    lax.fori_loop(0, K, fz, 0)

    def iz(j, carry):
        pltpu.sync_copy(rows, acc.at[pl.ds(s * RPT + j * K, K)])
        return carry
    lax.fori_loop(0, RPT // K, iz, 0)
    plsc.subcore_barrier()

    def chunk(i, carry):
        base = wid * EPW + i * K
        pltpu.sync_copy(src_hbm.at[pl.ds(base, K)], sidx)
        pltpu.sync_copy(dst_hbm.at[pl.ds(base, K)], didx)
        pltpu.async_copy(u_hbm.at[sidx], rows, sem).wait()
        pltpu.sync_copy(rows, acc.at[didx], add=True)
        return carry
    lax.fori_loop(0, NCHUNK, chunk, 0)

    plsc.subcore_barrier()
    pltpu.sync_copy(acc.at[pl.ds(s * RPT, RPT)], buf)
    pltpu.sync_copy(buf, out_hbm.at[pl.ds(c * NPAD + s * RPT, RPT)])


_R = 1000


def _u1_body(x_ref, w_ref, dinv_ref, o_ref):
    o_ref[...] = dinv_ref[...] * jnp.dot(
        x_ref[...], w_ref[...], preferred_element_type=jnp.float32)


def _u1(x, W1, dinv2):
    return pl.pallas_call(
        _u1_body,
        grid=(N // _R,),
        in_specs=[
            pl.BlockSpec((_R, DIN), lambda i: (i, 0)),
            pl.BlockSpec((DIN, DH), lambda i: (0, 0)),
            pl.BlockSpec((_R, 1), lambda i: (i, 0)),
        ],
        out_specs=pl.BlockSpec((_R, DH), lambda i: (i, 0)),
        out_shape=jax.ShapeDtypeStruct((N, DH), jnp.float32),
    )(x, W1, dinv2)


def _trans_body(a0_ref, a1_ref, u_ref, dinv_ref, b_ref, w_ref, o_ref):
    h = jnp.maximum(
        dinv_ref[...] * (a0_ref[...] + a1_ref[...] + u_ref[...]) + b_ref[...],
        0.0)
    o_ref[...] = dinv_ref[...] * jnp.dot(
        h, w_ref[...], preferred_element_type=jnp.float32)


def _trans(a0, a1, u, dinv2, b_row, W_next):
    return pl.pallas_call(
        _trans_body,
        grid=(N // _R,),
        in_specs=[
            pl.BlockSpec((_R, DH), lambda i: (i, 0)),
            pl.BlockSpec((_R, DH), lambda i: (i, 0)),
            pl.BlockSpec((_R, DH), lambda i: (i, 0)),
            pl.BlockSpec((_R, 1), lambda i: (i, 0)),
            pl.BlockSpec((1, DH), lambda i: (0, 0)),
            pl.BlockSpec((DH, DH), lambda i: (0, 0)),
        ],
        out_specs=pl.BlockSpec((_R, DH), lambda i: (i, 0)),
        out_shape=jax.ShapeDtypeStruct((N, DH), jnp.float32),
    )(a0, a1, u, dinv2, b_row, W_next)


def _final_body(a0_ref, a1_ref, u_ref, dinv_ref, b_ref, bf_ref, wl_ref,
                bl_ref, o_ref, sums, cnts):
    step = pl.program_id(0)

    @pl.when(step == 0)
    def _():
        sums[...] = jnp.zeros_like(sums)
        cnts[...] = jnp.zeros_like(cnts)

    h = dinv_ref[...] * (a0_ref[...] + a1_ref[...] + u_ref[...]) + b_ref[...]
    gid = lax.broadcasted_iota(jnp.float32, (1, G), 1)
    onehot = (bf_ref[...] == gid).astype(jnp.float32)
    dn = (((0,), (0,)), ((), ()))
    sums[...] += lax.dot_general(onehot, h, dn,
                                 preferred_element_type=jnp.float32)
    cnts[...] += lax.dot_general(onehot, jnp.ones((_R, 1), jnp.float32), dn,
                                 preferred_element_type=jnp.float32)

    @pl.when(step == pl.num_programs(0) - 1)
    def _():
        g = sums[...] / jnp.maximum(cnts[...], 1.0)
        o_ref[...] = jnp.dot(g, wl_ref[...],
                             preferred_element_type=jnp.float32) + bl_ref[...]


def _final(a0, a1, u, dinv2, b_row, batchf, Wl, bl2):
    return pl.pallas_call(
        _final_body,
        grid=(N // _R,),
        in_specs=[
            pl.BlockSpec((_R, DH), lambda i: (i, 0)),
            pl.BlockSpec((_R, DH), lambda i: (i, 0)),
            pl.BlockSpec((_R, DH), lambda i: (i, 0)),
            pl.BlockSpec((_R, 1), lambda i: (i, 0)),
            pl.BlockSpec((1, DH), lambda i: (0, 0)),
            pl.BlockSpec((_R, 1), lambda i: (i, 0)),
            pl.BlockSpec((DH, 1), lambda i: (0, 0)),
            pl.BlockSpec((1, 1), lambda i: (0, 0)),
        ],
        out_specs=pl.BlockSpec((G, 1), lambda i: (0, 0)),
        out_shape=jax.ShapeDtypeStruct((G, 1), jnp.float32),
        scratch_shapes=[
            pltpu.VMEM((G, DH), jnp.float32),
            pltpu.VMEM((G, 1), jnp.float32),
        ],
    )(a0, a1, u, dinv2, b_row, batchf, Wl, bl2)


def kernel(x, edge_index, batch, W1, b1, W2, b2, W3, b3, Wl, bl):
    src = edge_index[0].astype(jnp.int32)
    dst = edge_index[1].astype(jnp.int32)
    batchf = batch.astype(jnp.float32)[:, None]

    degp = _deg(dst)
    deg = degp.reshape(NC, NPAD).sum(0)[:N] + 1.0
    dinv2 = lax.rsqrt(deg)[:, None]

    u1 = _u1(x, W1, dinv2)
    a = _agg(u1, src, dst).reshape(NC, NPAD, DH)
    u2 = _trans(a[0, :N], a[1, :N], u1, dinv2, b1[None, :], W2)
    a = _agg(u2, src, dst).reshape(NC, NPAD, DH)
    u3 = _trans(a[0, :N], a[1, :N], u2, dinv2, b2[None, :], W3)
    a = _agg(u3, src, dst).reshape(NC, NPAD, DH)
    return _final(a[0, :N], a[1, :N], u3, dinv2, b3[None, :], batchf,
                  Wl, bl[None, :])


# speedup vs baseline: 12.7619x; 12.7619x over previous
import functools

import jax
import jax.numpy as jnp
from jax import lax
from jax.experimental import pallas as pl
from jax.experimental.pallas import tpu as pltpu
from jax.experimental.pallas import tpu_sc as plsc

N = 10000
NPAD = 10240
E = 320000
DIN = 128
DH = 64
G = 64
NC = 2
NS = 16
NW = NC * NS
EPW = E // NW
K = 80
NCHUNK = EPW // K
RPT = NPAD // NS

_MESH = dict(core_axis_name="c", subcore_axis_name="s")


@functools.partial(
    pl.kernel,
    out_type=jax.ShapeDtypeStruct((NC * NPAD,), jnp.float32),
    mesh=plsc.VectorSubcoreMesh(**_MESH),
    scratch_types=[
        pltpu.VMEM((K,), jnp.int32),
        pltpu.VMEM((K,), jnp.float32),
        pltpu.VMEM((RPT,), jnp.float32),
        pltpu.VMEM_SHARED((NPAD,), jnp.float32),
    ],
)
def _deg(dst_hbm, out_hbm, idx_v, ones_v, buf_v, acc):
    c = lax.axis_index("c")
    s = lax.axis_index("s")
    wid = s * NC + c

    def fill_ones(i, carry):
        ones_v[pl.ds(i * 16, 16)] = jnp.ones((16,), jnp.float32)
        return carry
    lax.fori_loop(0, K // 16, fill_ones, 0)

    def fill_zeros(i, carry):
        buf_v[pl.ds(i * 16, 16)] = jnp.zeros((16,), jnp.float32)
        return carry
    lax.fori_loop(0, RPT // 16, fill_zeros, 0)

    pltpu.sync_copy(buf_v, acc.at[pl.ds(s * RPT, RPT)])
    plsc.subcore_barrier()

    def chunk(i, carry):
        base = wid * EPW + i * K
        pltpu.sync_copy(dst_hbm.at[pl.ds(base, K)], idx_v)
        pltpu.sync_copy(ones_v, acc.at[idx_v], add=True)
        return carry
    lax.fori_loop(0, NCHUNK, chunk, 0)

    plsc.subcore_barrier()
    pltpu.sync_copy(acc.at[pl.ds(s * RPT, RPT)], buf_v)
    pltpu.sync_copy(buf_v, out_hbm.at[pl.ds(c * NPAD + s * RPT, RPT)])


@functools.partial(
    pl.kernel,
    out_type=jax.ShapeDtypeStruct((NC * NPAD, DH), jnp.float32),
    mesh=plsc.VectorSubcoreMesh(**_MESH),
    scratch_types=[
        pltpu.VMEM((K,), jnp.int32),
        pltpu.VMEM((K,), jnp.int32),
        pltpu.VMEM((K, DH), jnp.float32),
        pltpu.VMEM((RPT, DH), jnp.float32),
        pltpu.VMEM_SHARED((NPAD, DH), jnp.float32),
        pltpu.SemaphoreType.DMA,
    ],
    compiler_params=pltpu.CompilerParams(use_tc_tiling_on_sc=False),
)
def _agg(u_hbm, src_hbm, dst_hbm, out_hbm, sidx, didx, rows, buf, acc, sem):
    c = lax.axis_index("c")
    s = lax.axis_index("s")
    wid = s * NC + c

    def fz(i, carry):
        for j in range(DH // 16):
            rows[i, pl.ds(j * 16, 16)] = jnp.zeros((16,), jnp.float32)
        return carry
    lax.fori_loop(0, K, fz, 0)

    def iz(j, carry):
        pltpu.sync_copy(rows, acc.at[pl.ds(s * RPT + j * K, K)])
        return carry
    lax.fori_loop(0, RPT // K, iz, 0)
    plsc.subcore_barrier()

    def chunk(i, carry):
        base = wid * EPW + i * K
        pltpu.sync_copy(src_hbm.at[pl.ds(base, K)], sidx)
        pltpu.sync_copy(dst_hbm.at[pl.ds(base, K)], didx)
        pltpu.async_copy(u_hbm.at[sidx], rows, sem).wait()
        pltpu.sync_copy(rows, acc.at[didx], add=True)
        return carry
    lax.fori_loop(0, NCHUNK, chunk, 0)

    plsc.subcore_barrier()
    pltpu.sync_copy(acc.at[pl.ds(s * RPT, RPT)], buf)
    pltpu.sync_copy(buf, out_hbm.at[pl.ds(c * NPAD + s * RPT, RPT)])


_R = 1000


def _u1_body(x_ref, w_ref, dinv_ref, o_ref):
    o_ref[...] = dinv_ref[...] * jnp.dot(
        x_ref[...], w_ref[...], preferred_element_type=jnp.float32)


def _u1(x, W1, dinv2):
    return pl.pallas_call(
        _u1_body,
        grid=(N // _R,),
        in_specs=[
            pl.BlockSpec((_R, DIN), lambda i: (i, 0)),
            pl.BlockSpec((DIN, DH), lambda i: (0, 0)),
            pl.BlockSpec((_R, 1), lambda i: (i, 0)),
        ],
        out_specs=pl.BlockSpec((_R, DH), lambda i: (i, 0)),
        out_shape=jax.ShapeDtypeStruct((N, DH), jnp.float32),
    )(x, W1, dinv2)


def _trans_body(a0_ref, a1_ref, u_ref, dinv_ref, b_ref, w_ref, o_ref):
    h = jnp.maximum(
        dinv_ref[...] * (a0_ref[...] + a1_ref[...] + u_ref[...]) + b_ref[...],
        0.0)
    o_ref[...] = dinv_ref[...] * jnp.dot(
        h, w_ref[...], preferred_element_type=jnp.float32)


def _trans(a0, a1, u, dinv2, b_row, W_next):
    return pl.pallas_call(
        _trans_body,
        grid=(N // _R,),
        in_specs=[
            pl.BlockSpec((_R, DH), lambda i: (i, 0)),
            pl.BlockSpec((_R, DH), lambda i: (i, 0)),
            pl.BlockSpec((_R, DH), lambda i: (i, 0)),
            pl.BlockSpec((_R, 1), lambda i: (i, 0)),
            pl.BlockSpec((1, DH), lambda i: (0, 0)),
            pl.BlockSpec((DH, DH), lambda i: (0, 0)),
        ],
        out_specs=pl.BlockSpec((_R, DH), lambda i: (i, 0)),
        out_shape=jax.ShapeDtypeStruct((N, DH), jnp.float32),
    )(a0, a1, u, dinv2, b_row, W_next)


def _final_body(a0_ref, a1_ref, u_ref, dinv_ref, b_ref, bf_ref, wl_ref,
                bl_ref, o_ref, sums, cnts):
    step = pl.program_id(0)

    @pl.when(step == 0)
    def _():
        sums[...] = jnp.zeros_like(sums)
        cnts[...] = jnp.zeros_like(cnts)

    h = dinv_ref[...] * (a0_ref[...] + a1_ref[...] + u_ref[...]) + b_ref[...]
    gid = lax.broadcasted_iota(jnp.int32, (1, G), 1).astype(jnp.float32)
    onehot = (bf_ref[...] == gid).astype(jnp.float32)
    dn = (((0,), (0,)), ((), ()))
    sums[...] += lax.dot_general(onehot, h, dn,
                                 preferred_element_type=jnp.float32)
    cnts[...] += lax.dot_general(onehot, jnp.ones((_R, 1), jnp.float32), dn,
                                 preferred_element_type=jnp.float32)

    @pl.when(step == pl.num_programs(0) - 1)
    def _():
        g = sums[...] / jnp.maximum(cnts[...], 1.0)
        o_ref[...] = jnp.dot(g, wl_ref[...],
                             preferred_element_type=jnp.float32) + bl_ref[...]


def _final(a0, a1, u, dinv2, b_row, batchf, Wl, bl2):
    return pl.pallas_call(
        _final_body,
        grid=(N // _R,),
        in_specs=[
            pl.BlockSpec((_R, DH), lambda i: (i, 0)),
            pl.BlockSpec((_R, DH), lambda i: (i, 0)),
            pl.BlockSpec((_R, DH), lambda i: (i, 0)),
            pl.BlockSpec((_R, 1), lambda i: (i, 0)),
            pl.BlockSpec((1, DH), lambda i: (0, 0)),
            pl.BlockSpec((_R, 1), lambda i: (i, 0)),
            pl.BlockSpec((DH, 1), lambda i: (0, 0)),
            pl.BlockSpec((1, 1), lambda i: (0, 0)),
        ],
        out_specs=pl.BlockSpec((G, 1), lambda i: (0, 0)),
        out_shape=jax.ShapeDtypeStruct((G, 1), jnp.float32),
        scratch_shapes=[
            pltpu.VMEM((G, DH), jnp.float32),
            pltpu.VMEM((G, 1), jnp.float32),
        ],
    )(a0, a1, u, dinv2, b_row, batchf, Wl, bl2)


def kernel(x, edge_index, batch, W1, b1, W2, b2, W3, b3, Wl, bl):
    src = edge_index[0].astype(jnp.int32)
    dst = edge_index[1].astype(jnp.int32)
    batchf = batch.astype(jnp.float32)[:, None]

    degp = _deg(dst)
    deg = degp.reshape(NC, NPAD).sum(0)[:N] + 1.0
    dinv2 = lax.rsqrt(deg)[:, None]

    u1 = _u1(x, W1, dinv2)
    a = _agg(u1, src, dst).reshape(NC, NPAD, DH)
    u2 = _trans(a[0, :N], a[1, :N], u1, dinv2, b1[None, :], W2)
    a = _agg(u2, src, dst).reshape(NC, NPAD, DH)
    u3 = _trans(a[0, :N], a[1, :N], u2, dinv2, b2[None, :], W3)
    a = _agg(u3, src, dst).reshape(NC, NPAD, DH)
    return _final(a[0, :N], a[1, :N], u3, dinv2, b3[None, :], batchf,
                  Wl, bl[None, :])


# speedup vs baseline: 35.9934x; 2.8204x over previous
import functools

import jax
import jax.numpy as jnp
from jax import lax
from jax.experimental import pallas as pl
from jax.experimental.pallas import tpu as pltpu
from jax.experimental.pallas import tpu_sc as plsc

N = 10000
NPAD = 10240
E = 320000
DIN = 128
DH = 64
G = 64
NC = 2
NS = 16
NW = NC * NS
EPW = E // NW
KD = 80
CD = EPW // KD
KA = 125
CA = EPW // KA
NB = 4
RPT = NPAD // NS

_MESH = dict(core_axis_name="c", subcore_axis_name="s")


@functools.partial(
    pl.kernel,
    out_type=jax.ShapeDtypeStruct((NC * NPAD,), jnp.float32),
    mesh=plsc.VectorSubcoreMesh(**_MESH),
    scratch_types=[
        pltpu.VMEM((CD, KD), jnp.int32),
        pltpu.VMEM((KD,), jnp.float32),
        pltpu.VMEM((RPT,), jnp.float32),
        pltpu.VMEM_SHARED((NPAD,), jnp.float32),
        pltpu.SemaphoreType.DMA,
    ],
)
def _deg(dst_hbm, out_hbm, didx2, ones_v, buf_v, acc, sem):
    c = lax.axis_index("c")
    s = lax.axis_index("s")
    wid = s * NC + c

    def fill_ones(i, carry):
        ones_v[pl.ds(i * 16, 16)] = jnp.ones((16,), jnp.float32)
        return carry
    lax.fori_loop(0, KD // 16, fill_ones, 0)

    def fill_zeros(i, carry):
        buf_v[pl.ds(i * 16, 16)] = jnp.zeros((16,), jnp.float32)
        return carry
    lax.fori_loop(0, RPT // 16, fill_zeros, 0)

    pltpu.sync_copy(buf_v, acc.at[pl.ds(s * RPT, RPT)])
    pltpu.sync_copy(dst_hbm.at[wid], didx2)
    plsc.subcore_barrier()

    def chunk(i, carry):
        pltpu.async_copy(ones_v, acc.at[didx2.at[i]], sem, add=True)

        @pl.when(i >= 4)
        def _():
            pltpu.make_async_copy(ones_v, acc.at[didx2.at[0]], sem).wait()
        return carry
    lax.fori_loop(0, CD, chunk, 0)
    for _ in range(4):
        pltpu.make_async_copy(ones_v, acc.at[didx2.at[0]], sem).wait()

    plsc.subcore_barrier()
    pltpu.sync_copy(acc.at[pl.ds(s * RPT, RPT)], buf_v)
    pltpu.sync_copy(buf_v, out_hbm.at[pl.ds(c * NPAD + s * RPT, RPT)])


@functools.partial(
    pl.kernel,
    out_type=jax.ShapeDtypeStruct((NC * NPAD, DH), jnp.float32),
    mesh=plsc.VectorSubcoreMesh(**_MESH),
    scratch_types=[
        pltpu.VMEM((CA, KA), jnp.int32),
        pltpu.VMEM((CA, KA), jnp.int32),
        pltpu.VMEM((NB, KA, DH), jnp.float32),
        pltpu.VMEM_SHARED((NPAD, DH), jnp.float32),
        [pltpu.SemaphoreType.DMA] * NB,
        [pltpu.SemaphoreType.DMA] * NB,
    ],
    compiler_params=pltpu.CompilerParams(use_tc_tiling_on_sc=False),
)
def _agg(u_hbm, src_hbm, dst_hbm, out_hbm, sidx2, didx2, rows, acc,
         gsem, ssem):
    c = lax.axis_index("c")
    s = lax.axis_index("s")
    wid = s * NC + c

    def fz(i, carry):
        for j in range(DH // 16):
            rows[0, i, pl.ds(j * 16, 16)] = jnp.zeros((16,), jnp.float32)
        return carry
    lax.fori_loop(0, KA, fz, 0)

    pltpu.sync_copy(src_hbm.at[wid], sidx2)
    pltpu.sync_copy(dst_hbm.at[wid], didx2)

    def iz(j, carry):
        pltpu.sync_copy(rows.at[0], acc.at[pl.ds(s * RPT + j * KA, KA)])
        return carry
    lax.fori_loop(0, RPT // KA, iz, 0)
    pltpu.sync_copy(rows.at[0, pl.ds(0, 15)],
                    acc.at[pl.ds(s * RPT + (RPT // KA) * KA, 15)])
    for b in range(NB):
        pltpu.async_copy(u_hbm.at[sidx2.at[b]], rows.at[b], gsem[b])
    plsc.subcore_barrier()

    def round_(j, carry):
        for b in range(NB):
            i = j * NB + b
            pltpu.make_async_copy(
                u_hbm.at[sidx2.at[i]], rows.at[b], gsem[b]).wait()
            pltpu.async_copy(rows.at[b], acc.at[didx2.at[i]], ssem[b],
                             add=True)
        for b in range(NB):
            i = j * NB + b
            pltpu.make_async_copy(
                rows.at[b], acc.at[didx2.at[i]], ssem[b]).wait()

            @pl.when(j < CA // NB - 1)
            def _():
                pltpu.async_copy(
                    u_hbm.at[sidx2.at[i + NB]], rows.at[b], gsem[b])
        return carry
    lax.fori_loop(0, CA // NB, round_, 0)

    plsc.subcore_barrier()
    def ro(j, carry):
        b = 0
        pltpu.sync_copy(acc.at[pl.ds(s * RPT + j * KA, KA)], rows.at[b])
        pltpu.sync_copy(rows.at[b],
                        out_hbm.at[pl.ds(c * NPAD + s * RPT + j * KA, KA)])
        return carry
    lax.fori_loop(0, RPT // KA, ro, 0)
    rem = RPT - (RPT // KA) * KA
    pltpu.sync_copy(acc.at[pl.ds(s * RPT + (RPT // KA) * KA, rem)],
                    rows.at[1, pl.ds(0, rem)])
    pltpu.sync_copy(rows.at[1, pl.ds(0, rem)],
                    out_hbm.at[pl.ds(c * NPAD + s * RPT + (RPT // KA) * KA,
                                     rem)])


_R = 1000


def _u1_body(x_ref, w_ref, dinv_ref, o_ref):
    o_ref[...] = dinv_ref[...] * jnp.dot(
        x_ref[...], w_ref[...], preferred_element_type=jnp.float32)


def _u1(x, W1, dinv2):
    return pl.pallas_call(
        _u1_body,
        grid=(N // _R,),
        in_specs=[
            pl.BlockSpec((_R, DIN), lambda i: (i, 0)),
            pl.BlockSpec((DIN, DH), lambda i: (0, 0)),
            pl.BlockSpec((_R, 1), lambda i: (i, 0)),
        ],
        out_specs=pl.BlockSpec((_R, DH), lambda i: (i, 0)),
        out_shape=jax.ShapeDtypeStruct((N, DH), jnp.float32),
    )(x, W1, dinv2)


def _trans_body(a0_ref, a1_ref, u_ref, dinv_ref, b_ref, w_ref, o_ref):
    h = jnp.maximum(
        dinv_ref[...] * (a0_ref[...] + a1_ref[...] + u_ref[...]) + b_ref[...],
        0.0)
    o_ref[...] = dinv_ref[...] * jnp.dot(
        h, w_ref[...], preferred_element_type=jnp.float32)


def _trans(a0, a1, u, dinv2, b_row, W_next):
    return pl.pallas_call(
        _trans_body,
        grid=(N // _R,),
        in_specs=[
            pl.BlockSpec((_R, DH), lambda i: (i, 0)),
            pl.BlockSpec((_R, DH), lambda i: (i, 0)),
            pl.BlockSpec((_R, DH), lambda i: (i, 0)),
            pl.BlockSpec((_R, 1), lambda i: (i, 0)),
            pl.BlockSpec((1, DH), lambda i: (0, 0)),
            pl.BlockSpec((DH, DH), lambda i: (0, 0)),
        ],
        out_specs=pl.BlockSpec((_R, DH), lambda i: (i, 0)),
        out_shape=jax.ShapeDtypeStruct((N, DH), jnp.float32),
    )(a0, a1, u, dinv2, b_row, W_next)


def _final_body(a0_ref, a1_ref, u_ref, dinv_ref, b_ref, bf_ref, wl_ref,
                bl_ref, o_ref, sums, cnts):
    step = pl.program_id(0)

    @pl.when(step == 0)
    def _():
        sums[...] = jnp.zeros_like(sums)
        cnts[...] = jnp.zeros_like(cnts)

    h = dinv_ref[...] * (a0_ref[...] + a1_ref[...] + u_ref[...]) + b_ref[...]
    gid = lax.broadcasted_iota(jnp.int32, (1, G), 1).astype(jnp.float32)
    onehot = (bf_ref[...] == gid).astype(jnp.float32)
    dn = (((0,), (0,)), ((), ()))
    sums[...] += lax.dot_general(onehot, h, dn,
                                 preferred_element_type=jnp.float32)
    cnts[...] += lax.dot_general(onehot, jnp.ones((_R, 1), jnp.float32), dn,
                                 preferred_element_type=jnp.float32)

    @pl.when(step == pl.num_programs(0) - 1)
    def _():
        g = sums[...] / jnp.maximum(cnts[...], 1.0)
        o_ref[...] = jnp.dot(g, wl_ref[...],
                             preferred_element_type=jnp.float32) + bl_ref[...]


def _final(a0, a1, u, dinv2, b_row, batchf, Wl, bl2):
    return pl.pallas_call(
        _final_body,
        grid=(N // _R,),
        in_specs=[
            pl.BlockSpec((_R, DH), lambda i: (i, 0)),
            pl.BlockSpec((_R, DH), lambda i: (i, 0)),
            pl.BlockSpec((_R, DH), lambda i: (i, 0)),
            pl.BlockSpec((_R, 1), lambda i: (i, 0)),
            pl.BlockSpec((1, DH), lambda i: (0, 0)),
            pl.BlockSpec((_R, 1), lambda i: (i, 0)),
            pl.BlockSpec((DH, 1), lambda i: (0, 0)),
            pl.BlockSpec((1, 1), lambda i: (0, 0)),
        ],
        out_specs=pl.BlockSpec((G, 1), lambda i: (0, 0)),
        out_shape=jax.ShapeDtypeStruct((G, 1), jnp.float32),
        scratch_shapes=[
            pltpu.VMEM((G, DH), jnp.float32),
            pltpu.VMEM((G, 1), jnp.float32),
        ],
    )(a0, a1, u, dinv2, b_row, batchf, Wl, bl2)


def kernel(x, edge_index, batch, W1, b1, W2, b2, W3, b3, Wl, bl):
    src = edge_index[0].astype(jnp.int32)
    dst = edge_index[1].astype(jnp.int32)
    batchf = batch.astype(jnp.float32)[:, None]

    dst_d = dst.reshape(NW, CD, KD)
    src_a = src.reshape(NW, CA, KA)
    dst_a = dst.reshape(NW, CA, KA)

    degp = _deg(dst_d)
    deg = degp.reshape(NC, NPAD).sum(0)[:N] + 1.0
    dinv2 = lax.rsqrt(deg)[:, None]

    u1 = _u1(x, W1, dinv2)
    a = _agg(u1, src_a, dst_a).reshape(NC, NPAD, DH)
    u2 = _trans(a[0, :N], a[1, :N], u1, dinv2, b1[None, :], W2)
    a = _agg(u2, src_a, dst_a).reshape(NC, NPAD, DH)
    u3 = _trans(a[0, :N], a[1, :N], u2, dinv2, b2[None, :], W3)
    a = _agg(u3, src_a, dst_a).reshape(NC, NPAD, DH)
    return _final(a[0, :N], a[1, :N], u3, dinv2, b3[None, :], batchf,
                  Wl, bl[None, :])


# speedup vs baseline: 38.1226x; 1.0592x over previous
import functools

import jax
import jax.numpy as jnp
from jax import lax
from jax.experimental import pallas as pl
from jax.experimental.pallas import tpu as pltpu
from jax.experimental.pallas import tpu_sc as plsc

N = 10000
NPAD = 10240
E = 320000
DIN = 128
DH = 64
G = 64
NC = 2
NS = 16
NW = NC * NS
EPW = E // NW
KA = 125
CA = EPW // KA
NB = 8
RPT = NPAD // NS

_MESH = dict(core_axis_name="c", subcore_axis_name="s")


@functools.partial(
    pl.kernel,
    out_type=jax.ShapeDtypeStruct((NC * NPAD,), jnp.float32),
    mesh=plsc.VectorSubcoreMesh(**_MESH),
    scratch_types=[
        pltpu.VMEM((CA, KA), jnp.int32),
        pltpu.VMEM((128,), jnp.float32),
        pltpu.VMEM((RPT,), jnp.float32),
        pltpu.VMEM_SHARED((NPAD,), jnp.float32),
        pltpu.SemaphoreType.DMA,
    ],
)
def _deg(dst_hbm, out_hbm, didx2, ones_v, buf_v, acc, sem):
    c = lax.axis_index("c")
    s = lax.axis_index("s")
    wid = s * NC + c

    def fill_ones(i, carry):
        ones_v[pl.ds(i * 16, 16)] = jnp.ones((16,), jnp.float32)
        return carry
    lax.fori_loop(0, 128 // 16, fill_ones, 0)

    def fill_zeros(i, carry):
        buf_v[pl.ds(i * 16, 16)] = jnp.zeros((16,), jnp.float32)
        return carry
    lax.fori_loop(0, RPT // 16, fill_zeros, 0)

    pltpu.sync_copy(buf_v, acc.at[pl.ds(s * RPT, RPT)])
    pltpu.sync_copy(dst_hbm.at[wid], didx2)
    plsc.subcore_barrier()

    def chunk(i, carry):
        pltpu.async_copy(ones_v.at[pl.ds(0, KA)], acc.at[didx2.at[i]], sem,
                         add=True)

        @pl.when(i >= 4)
        def _():
            pltpu.make_async_copy(
                ones_v.at[pl.ds(0, KA)], acc.at[didx2.at[0]], sem).wait()
        return carry
    lax.fori_loop(0, CA, chunk, 0)
    for _ in range(4):
        pltpu.make_async_copy(
            ones_v.at[pl.ds(0, KA)], acc.at[didx2.at[0]], sem).wait()

    plsc.subcore_barrier()
    pltpu.sync_copy(acc.at[pl.ds(s * RPT, RPT)], buf_v)
    pltpu.sync_copy(buf_v, out_hbm.at[pl.ds(c * NPAD + s * RPT, RPT)])


@functools.partial(
    pl.kernel,
    out_type=jax.ShapeDtypeStruct((NC * NPAD, DH), jnp.float32),
    mesh=plsc.VectorSubcoreMesh(**_MESH),
    scratch_types=[
        pltpu.VMEM((CA, KA), jnp.int32),
        pltpu.VMEM((CA, KA), jnp.int32),
        pltpu.VMEM((NB, KA, DH), jnp.float32),
        pltpu.VMEM_SHARED((NPAD, DH), jnp.float32),
        [pltpu.SemaphoreType.DMA] * NB,
        [pltpu.SemaphoreType.DMA] * NB,
    ],
    compiler_params=pltpu.CompilerParams(use_tc_tiling_on_sc=False),
)
def _agg(u_hbm, src_hbm, dst_hbm, out_hbm, sidx2, didx2, rows, acc,
         gsem, ssem):
    c = lax.axis_index("c")
    s = lax.axis_index("s")
    wid = s * NC + c

    def fz(i, carry):
        for j in range(DH // 16):
            rows[0, i, pl.ds(j * 16, 16)] = jnp.zeros((16,), jnp.float32)
        return carry
    lax.fori_loop(0, KA, fz, 0)

    pltpu.sync_copy(src_hbm.at[wid], sidx2)
    pltpu.sync_copy(dst_hbm.at[wid], didx2)

    def iz(j, carry):
        pltpu.sync_copy(rows.at[0], acc.at[pl.ds(s * RPT + j * KA, KA)])
        return carry
    lax.fori_loop(0, RPT // KA, iz, 0)
    pltpu.sync_copy(rows.at[0, pl.ds(0, 15)],
                    acc.at[pl.ds(s * RPT + (RPT // KA) * KA, 15)])
    for b in range(NB):
        pltpu.async_copy(u_hbm.at[sidx2.at[b]], rows.at[b], gsem[b])
    plsc.subcore_barrier()

    def round_(j, carry):
        for b in range(NB):
            i = j * NB + b
            pltpu.make_async_copy(
                u_hbm.at[sidx2.at[i]], rows.at[b], gsem[b]).wait()
            pltpu.async_copy(rows.at[b], acc.at[didx2.at[i]], ssem[b],
                             add=True)
        for b in range(NB):
            i = j * NB + b
            pltpu.make_async_copy(
                rows.at[b], acc.at[didx2.at[i]], ssem[b]).wait()

            @pl.when(j < CA // NB - 1)
            def _():
                pltpu.async_copy(
                    u_hbm.at[sidx2.at[i + NB]], rows.at[b], gsem[b])
        return carry
    lax.fori_loop(0, CA // NB, round_, 0)

    plsc.subcore_barrier()
    def ro(j, carry):
        pltpu.sync_copy(acc.at[pl.ds(s * RPT + j * KA, KA)], rows.at[0])
        pltpu.sync_copy(rows.at[0],
                        out_hbm.at[pl.ds(c * NPAD + s * RPT + j * KA, KA)])
        return carry
    lax.fori_loop(0, RPT // KA, ro, 0)
    rem = RPT - (RPT // KA) * KA
    pltpu.sync_copy(acc.at[pl.ds(s * RPT + (RPT // KA) * KA, rem)],
                    rows.at[1, pl.ds(0, rem)])
    pltpu.sync_copy(rows.at[1, pl.ds(0, rem)],
                    out_hbm.at[pl.ds(c * NPAD + s * RPT + (RPT // KA) * KA,
                                     rem)])


_R = 640
_GRID = NPAD // _R
_A1 = NPAD // _R


def _u1_body(x_ref, w_ref, dinv_ref, o_ref):
    o_ref[...] = dinv_ref[...] * jnp.dot(
        x_ref[...], w_ref[...], preferred_element_type=jnp.float32)


def _u1(x, W1, dinv2):
    return pl.pallas_call(
        _u1_body,
        grid=(_GRID,),
        in_specs=[
            pl.BlockSpec((_R, DIN), lambda i: (i, 0)),
            pl.BlockSpec((DIN, DH), lambda i: (0, 0)),
            pl.BlockSpec((_R, 1), lambda i: (i, 0)),
        ],
        out_specs=pl.BlockSpec((_R, DH), lambda i: (i, 0)),
        out_shape=jax.ShapeDtypeStruct((N, DH), jnp.float32),
    )(x, W1, dinv2)


def _trans_body(a0_ref, a1_ref, u_ref, dinv_ref, b_ref, w_ref, o_ref):
    h = jnp.maximum(
        dinv_ref[...] * (a0_ref[...] + a1_ref[...] + u_ref[...]) + b_ref[...],
        0.0)
    o_ref[...] = dinv_ref[...] * jnp.dot(
        h, w_ref[...], preferred_element_type=jnp.float32)


def _trans(agg, u, dinv2, b_row, W_next):
    return pl.pallas_call(
        _trans_body,
        grid=(_GRID,),
        in_specs=[
            pl.BlockSpec((_R, DH), lambda i: (i, 0)),
            pl.BlockSpec((_R, DH), lambda i: (_A1 + i, 0)),
            pl.BlockSpec((_R, DH), lambda i: (i, 0)),
            pl.BlockSpec((_R, 1), lambda i: (i, 0)),
            pl.BlockSpec((1, DH), lambda i: (0, 0)),
            pl.BlockSpec((DH, DH), lambda i: (0, 0)),
        ],
        out_specs=pl.BlockSpec((_R, DH), lambda i: (i, 0)),
        out_shape=jax.ShapeDtypeStruct((N, DH), jnp.float32),
    )(agg, agg, u, dinv2, b_row, W_next)


def _final_body(a0_ref, a1_ref, u_ref, dinv_ref, b_ref, bf_ref, wl_ref,
                bl_ref, o_ref, sums, cnts):
    step = pl.program_id(0)

    @pl.when(step == 0)
    def _():
        sums[...] = jnp.zeros_like(sums)
        cnts[...] = jnp.zeros_like(cnts)

    rid = lax.broadcasted_iota(jnp.int32, (_R, 1), 0) + step * _R
    valid = rid < N
    h = dinv_ref[...] * (a0_ref[...] + a1_ref[...] + u_ref[...]) + b_ref[...]
    h = jnp.where(valid, h, 0.0)
    gid = lax.broadcasted_iota(jnp.int32, (1, G), 1).astype(jnp.float32)
    onehot = jnp.where(valid, (bf_ref[...] == gid).astype(jnp.float32), 0.0)
    dn = (((0,), (0,)), ((), ()))
    sums[...] += lax.dot_general(onehot, h, dn,
                                 preferred_element_type=jnp.float32)
    cnts[...] += lax.dot_general(onehot, jnp.ones((_R, 1), jnp.float32), dn,
                                 preferred_element_type=jnp.float32)

    @pl.when(step == pl.num_programs(0) - 1)
    def _():
        g = sums[...] / jnp.maximum(cnts[...], 1.0)
        o_ref[...] = jnp.dot(g, wl_ref[...],
                             preferred_element_type=jnp.float32) + bl_ref[...]


def _final(agg, u, dinv2, b_row, batchf, Wl, bl2):
    return pl.pallas_call(
        _final_body,
        grid=(_GRID,),
        in_specs=[
            pl.BlockSpec((_R, DH), lambda i: (i, 0)),
            pl.BlockSpec((_R, DH), lambda i: (_A1 + i, 0)),
            pl.BlockSpec((_R, DH), lambda i: (i, 0)),
            pl.BlockSpec((_R, 1), lambda i: (i, 0)),
            pl.BlockSpec((1, DH), lambda i: (0, 0)),
            pl.BlockSpec((_R, 1), lambda i: (i, 0)),
            pl.BlockSpec((DH, 1), lambda i: (0, 0)),
            pl.BlockSpec((1, 1), lambda i: (0, 0)),
        ],
        out_specs=pl.BlockSpec((G, 1), lambda i: (0, 0)),
        out_shape=jax.ShapeDtypeStruct((G, 1), jnp.float32),
        scratch_shapes=[
            pltpu.VMEM((G, DH), jnp.float32),
            pltpu.VMEM((G, 1), jnp.float32),
        ],
    )(agg, agg, u, dinv2, b_row, batchf, Wl, bl2)


def kernel(x, edge_index, batch, W1, b1, W2, b2, W3, b3, Wl, bl):
    src_a = edge_index[0].astype(jnp.int32).reshape(NW, CA, KA)
    dst_a = edge_index[1].astype(jnp.int32).reshape(NW, CA, KA)
    batchf = batch.astype(jnp.float32)[:, None]

    degp = _deg(dst_a)
    deg = degp.reshape(NC, NPAD).sum(0)[:N] + 1.0
    dinv2 = lax.rsqrt(deg)[:, None]

    u1 = _u1(x, W1, dinv2)
    a = _agg(u1, src_a, dst_a)
    u2 = _trans(a, u1, dinv2, b1[None, :], W2)
    a = _agg(u2, src_a, dst_a)
    u3 = _trans(a, u2, dinv2, b2[None, :], W3)
    a = _agg(u3, src_a, dst_a)
    return _final(a, u3, dinv2, b3[None, :], batchf, Wl, bl[None, :])


# speedup vs baseline: 39.6926x; 1.0412x over previous
import functools

import jax
import jax.numpy as jnp
from jax import lax
from jax.experimental import pallas as pl
from jax.experimental.pallas import tpu as pltpu
from jax.experimental.pallas import tpu_sc as plsc

N = 10000
NPAD = 10240
E = 320000
DIN = 128
DH = 64
G = 64
NC = 2
NS = 16
NW = NC * NS
EPW = E // NW
KA = 125
CA = EPW // KA
NB = 8
RPT = NPAD // NS

_MESH = dict(core_axis_name="c", subcore_axis_name="s")


@functools.partial(
    pl.kernel,
    out_type=jax.ShapeDtypeStruct((NC * NPAD,), jnp.float32),
    mesh=plsc.VectorSubcoreMesh(**_MESH),
    scratch_types=[
        pltpu.VMEM((CA, KA), jnp.int32),
        pltpu.VMEM((128,), jnp.float32),
        pltpu.VMEM((RPT,), jnp.float32),
        pltpu.VMEM_SHARED((NPAD,), jnp.float32),
        pltpu.SemaphoreType.DMA,
    ],
)
def _deg(dst_hbm, out_hbm, didx2, ones_v, buf_v, acc, sem):
    c = lax.axis_index("c")
    s = lax.axis_index("s")
    wid = s * NC + c

    def fill_ones(i, carry):
        ones_v[pl.ds(i * 16, 16)] = jnp.ones((16,), jnp.float32)
        return carry
    lax.fori_loop(0, 128 // 16, fill_ones, 0)

    def fill_zeros(i, carry):
        buf_v[pl.ds(i * 16, 16)] = jnp.zeros((16,), jnp.float32)
        return carry
    lax.fori_loop(0, RPT // 16, fill_zeros, 0)

    pltpu.sync_copy(buf_v, acc.at[pl.ds(s * RPT, RPT)])
    pltpu.sync_copy(dst_hbm.at[wid], didx2)
    plsc.subcore_barrier()

    def chunk(i, carry):
        pltpu.async_copy(ones_v.at[pl.ds(0, KA)], acc.at[didx2.at[i]], sem,
                         add=True)

        @pl.when(i >= 4)
        def _():
            pltpu.make_async_copy(
                ones_v.at[pl.ds(0, KA)], acc.at[didx2.at[0]], sem).wait()
        return carry
    lax.fori_loop(0, CA, chunk, 0)
    for _ in range(4):
        pltpu.make_async_copy(
            ones_v.at[pl.ds(0, KA)], acc.at[didx2.at[0]], sem).wait()

    plsc.subcore_barrier()
    pltpu.sync_copy(acc.at[pl.ds(s * RPT, RPT)], buf_v)
    pltpu.sync_copy(buf_v, out_hbm.at[pl.ds(c * NPAD + s * RPT, RPT)])


@functools.partial(
    pl.kernel,
    out_type=jax.ShapeDtypeStruct((NC * NPAD, DH), jnp.float32),
    mesh=plsc.VectorSubcoreMesh(**_MESH),
    scratch_types=[
        pltpu.VMEM((CA, KA), jnp.int32),
        pltpu.VMEM((CA, KA), jnp.int32),
        pltpu.VMEM((NB, KA, DH), jnp.float32),
        pltpu.VMEM_SHARED((NPAD, DH), jnp.float32),
        [pltpu.SemaphoreType.DMA] * NB,
        [pltpu.SemaphoreType.DMA] * NB,
    ],
    compiler_params=pltpu.CompilerParams(use_tc_tiling_on_sc=False),
)
def _agg(u_hbm, src_hbm, dst_hbm, out_hbm, sidx2, didx2, rows, acc,
         gsem, ssem):
    c = lax.axis_index("c")
    s = lax.axis_index("s")
    wid = s * NC + c

    def fz(i, carry):
        for j in range(DH // 16):
            rows[0, i, pl.ds(j * 16, 16)] = jnp.zeros((16,), jnp.float32)
        return carry
    lax.fori_loop(0, KA, fz, 0)

    pltpu.sync_copy(src_hbm.at[wid], sidx2)
    pltpu.sync_copy(dst_hbm.at[wid], didx2)

    def iz(j, carry):
        pltpu.sync_copy(rows.at[0], acc.at[pl.ds(s * RPT + j * KA, KA)])
        return carry
    lax.fori_loop(0, RPT // KA, iz, 0)
    pltpu.sync_copy(rows.at[0, pl.ds(0, 15)],
                    acc.at[pl.ds(s * RPT + (RPT // KA) * KA, 15)])
    for b in range(NB):
        pltpu.async_copy(u_hbm.at[sidx2.at[b]], rows.at[b], gsem[b])
    plsc.subcore_barrier()

    def round_(j, carry):
        for b in range(NB):
            i = j * NB + b
            pltpu.make_async_copy(
                u_hbm.at[sidx2.at[i]], rows.at[b], gsem[b]).wait()
            pltpu.async_copy(rows.at[b], acc.at[didx2.at[i]], ssem[b],
                             add=True)
        for b in range(NB):
            i = j * NB + b
            pltpu.make_async_copy(
                rows.at[b], acc.at[didx2.at[i]], ssem[b]).wait()

            @pl.when(j < CA // NB - 1)
            def _():
                pltpu.async_copy(
                    u_hbm.at[sidx2.at[i + NB]], rows.at[b], gsem[b])
        return carry
    lax.fori_loop(0, CA // NB, round_, 0)

    plsc.subcore_barrier()
    def ro(j, carry):
        pltpu.sync_copy(acc.at[pl.ds(s * RPT + j * KA, KA)], rows.at[0])
        pltpu.sync_copy(rows.at[0],
                        out_hbm.at[pl.ds(c * NPAD + s * RPT + j * KA, KA)])
        return carry
    lax.fori_loop(0, RPT // KA, ro, 0)
    rem = RPT - (RPT // KA) * KA
    pltpu.sync_copy(acc.at[pl.ds(s * RPT + (RPT // KA) * KA, rem)],
                    rows.at[1, pl.ds(0, rem)])
    pltpu.sync_copy(rows.at[1, pl.ds(0, rem)],
                    out_hbm.at[pl.ds(c * NPAD + s * RPT + (RPT // KA) * KA,
                                     rem)])


_R = 1024
_GRID = NPAD // _R
_A1 = NPAD // _R


def _u1_body(x_ref, w_ref, dinv_ref, o_ref):
    o_ref[...] = dinv_ref[...] * jnp.dot(
        x_ref[...], w_ref[...], preferred_element_type=jnp.float32)


def _u1(x, W1, dinv2):
    return pl.pallas_call(
        _u1_body,
        grid=(_GRID,),
        in_specs=[
            pl.BlockSpec((_R, DIN), lambda i: (i, 0)),
            pl.BlockSpec((DIN, DH), lambda i: (0, 0)),
            pl.BlockSpec((_R, 1), lambda i: (i, 0)),
        ],
        out_specs=pl.BlockSpec((_R, DH), lambda i: (i, 0)),
        out_shape=jax.ShapeDtypeStruct((N, DH), jnp.float32),
    )(x, W1, dinv2)


def _trans_body(a0_ref, a1_ref, u_ref, dinv_ref, b_ref, w_ref, o_ref):
    h = jnp.maximum(
        dinv_ref[...] * (a0_ref[...] + a1_ref[...] + u_ref[...]) + b_ref[...],
        0.0)
    o_ref[...] = dinv_ref[...] * jnp.dot(
        h, w_ref[...], preferred_element_type=jnp.float32)


def _trans(agg, u, dinv2, b_row, W_next):
    return pl.pallas_call(
        _trans_body,
        grid=(_GRID,),
        in_specs=[
            pl.BlockSpec((_R, DH), lambda i: (i, 0)),
            pl.BlockSpec((_R, DH), lambda i: (_A1 + i, 0)),
            pl.BlockSpec((_R, DH), lambda i: (i, 0)),
            pl.BlockSpec((_R, 1), lambda i: (i, 0)),
            pl.BlockSpec((1, DH), lambda i: (0, 0)),
            pl.BlockSpec((DH, DH), lambda i: (0, 0)),
        ],
        out_specs=pl.BlockSpec((_R, DH), lambda i: (i, 0)),
        out_shape=jax.ShapeDtypeStruct((N, DH), jnp.float32),
    )(agg, agg, u, dinv2, b_row, W_next)


def _final_body(a0_ref, a1_ref, u_ref, dinv_ref, b_ref, bf_ref, wl_ref,
                bl_ref, o_ref, sums, cnts):
    step = pl.program_id(0)

    @pl.when(step == 0)
    def _():
        sums[...] = jnp.zeros_like(sums)
        cnts[...] = jnp.zeros_like(cnts)

    rid = lax.broadcasted_iota(jnp.int32, (_R, 1), 0) + step * _R
    valid = rid < N
    h = dinv_ref[...] * (a0_ref[...] + a1_ref[...] + u_ref[...]) + b_ref[...]
    h = jnp.where(valid, h, 0.0)
    gid = lax.broadcasted_iota(jnp.int32, (1, G), 1).astype(jnp.float32)
    onehot = jnp.where(valid, (bf_ref[...] == gid).astype(jnp.float32), 0.0)
    dn = (((0,), (0,)), ((), ()))
    sums[...] += lax.dot_general(onehot, h, dn,
                                 preferred_element_type=jnp.float32)
    cnts[...] += lax.dot_general(onehot, jnp.ones((_R, 1), jnp.float32), dn,
                                 preferred_element_type=jnp.float32)

    @pl.when(step == pl.num_programs(0) - 1)
    def _():
        g = sums[...] / jnp.maximum(cnts[...], 1.0)
        o_ref[...] = jnp.dot(g, wl_ref[...],
                             preferred_element_type=jnp.float32) + bl_ref[...]


def _final(agg, u, dinv2, b_row, batchf, Wl, bl2):
    return pl.pallas_call(
        _final_body,
        grid=(_GRID,),
        in_specs=[
            pl.BlockSpec((_R, DH), lambda i: (i, 0)),
            pl.BlockSpec((_R, DH), lambda i: (_A1 + i, 0)),
            pl.BlockSpec((_R, DH), lambda i: (i, 0)),
            pl.BlockSpec((_R, 1), lambda i: (i, 0)),
            pl.BlockSpec((1, DH), lambda i: (0, 0)),
            pl.BlockSpec((_R, 1), lambda i: (i, 0)),
            pl.BlockSpec((DH, 1), lambda i: (0, 0)),
            pl.BlockSpec((1, 1), lambda i: (0, 0)),
        ],
        out_specs=pl.BlockSpec((G, 1), lambda i: (0, 0)),
        out_shape=jax.ShapeDtypeStruct((G, 1), jnp.float32),
        scratch_shapes=[
            pltpu.VMEM((G, DH), jnp.float32),
            pltpu.VMEM((G, 1), jnp.float32),
        ],
    )(agg, agg, u, dinv2, b_row, batchf, Wl, bl2)


def kernel(x, edge_index, batch, W1, b1, W2, b2, W3, b3, Wl, bl):
    src_a = edge_index[0].astype(jnp.int32).reshape(NW, CA, KA)
    dst_a = edge_index[1].astype(jnp.int32).reshape(NW, CA, KA)
    batchf = batch.astype(jnp.float32)[:, None]

    degp = _deg(dst_a)
    deg = degp.reshape(NC, NPAD).sum(0)[:N] + 1.0
    dinv2 = lax.rsqrt(deg)[:, None]

    u1 = _u1(x, W1, dinv2)
    a = _agg(u1, src_a, dst_a)
    u2 = _trans(a, u1, dinv2, b1[None, :], W2)
    a = _agg(u2, src_a, dst_a)
    u3 = _trans(a, u2, dinv2, b2[None, :], W3)
    a = _agg(u3, src_a, dst_a)
    return _final(a, u3, dinv2, b3[None, :], batchf, Wl, bl[None, :])


# speedup vs baseline: 46.3232x; 1.1670x over previous
import functools

import jax
import jax.numpy as jnp
from jax import lax
from jax.experimental import pallas as pl
from jax.experimental.pallas import tpu as pltpu
from jax.experimental.pallas import tpu_sc as plsc

N = 10000
NPAD = 10240
E = 320000
DIN = 128
DH = 64
G = 64
NC = 2
NS = 16
NW = NC * NS
EPW = E // NW
KA = 125
CA = EPW // KA
NB = 8
RPT = NPAD // NS

_MESH = dict(core_axis_name="c", subcore_axis_name="s")


@functools.partial(
    pl.kernel,
    out_type=jax.ShapeDtypeStruct((NC * NPAD,), jnp.float32),
    mesh=plsc.VectorSubcoreMesh(**_MESH),
    scratch_types=[
        pltpu.VMEM((CA, KA), jnp.int32),
        pltpu.VMEM((128,), jnp.float32),
        pltpu.VMEM((RPT,), jnp.float32),
        pltpu.VMEM_SHARED((NPAD,), jnp.float32),
        pltpu.SemaphoreType.DMA,
    ],
)
def _deg(dst_hbm, out_hbm, didx2, ones_v, buf_v, acc, sem):
    c = lax.axis_index("c")
    s = lax.axis_index("s")
    wid = s * NC + c

    def fill_ones(i, carry):
        ones_v[pl.ds(i * 16, 16)] = jnp.ones((16,), jnp.float32)
        return carry
    lax.fori_loop(0, 128 // 16, fill_ones, 0)

    def fill_zeros(i, carry):
        buf_v[pl.ds(i * 16, 16)] = jnp.zeros((16,), jnp.float32)
        return carry
    lax.fori_loop(0, RPT // 16, fill_zeros, 0)

    pltpu.sync_copy(buf_v, acc.at[pl.ds(s * RPT, RPT)])
    pltpu.sync_copy(dst_hbm.at[wid], didx2)
    plsc.subcore_barrier()

    def chunk(i, carry):
        pltpu.async_copy(ones_v.at[pl.ds(0, KA)], acc.at[didx2.at[i]], sem,
                         add=True)

        @pl.when(i >= 4)
        def _():
            pltpu.make_async_copy(
                ones_v.at[pl.ds(0, KA)], acc.at[didx2.at[0]], sem).wait()
        return carry
    lax.fori_loop(0, CA, chunk, 0)
    for _ in range(4):
        pltpu.make_async_copy(
            ones_v.at[pl.ds(0, KA)], acc.at[didx2.at[0]], sem).wait()

    plsc.subcore_barrier()
    pltpu.sync_copy(acc.at[pl.ds(s * RPT, RPT)], buf_v)
    pltpu.sync_copy(buf_v, out_hbm.at[pl.ds(c * NPAD + s * RPT, RPT)])


@functools.partial(
    pl.kernel,
    out_type=jax.ShapeDtypeStruct((NC * NPAD, DH), jnp.float32),
    mesh=plsc.VectorSubcoreMesh(**_MESH),
    scratch_types=[
        pltpu.VMEM((CA, KA), jnp.int32),
        pltpu.VMEM((CA, KA), jnp.int32),
        pltpu.VMEM((NB, KA, DH), jnp.float32),
        pltpu.VMEM_SHARED((NPAD, DH), jnp.float32),
        [pltpu.SemaphoreType.DMA] * NB,
        [pltpu.SemaphoreType.DMA] * NB,
    ],
    compiler_params=pltpu.CompilerParams(use_tc_tiling_on_sc=False),
)
def _agg(u_hbm, src_hbm, dst_hbm, out_hbm, sidx2, didx2, rows, acc,
         gsem, ssem):
    c = lax.axis_index("c")
    s = lax.axis_index("s")
    wid = s * NC + c

    def fz(i, carry):
        for j in range(DH // 16):
            rows[0, i, pl.ds(j * 16, 16)] = jnp.zeros((16,), jnp.float32)
        return carry
    lax.fori_loop(0, KA, fz, 0)

    pltpu.sync_copy(src_hbm.at[wid], sidx2)
    pltpu.sync_copy(dst_hbm.at[wid], didx2)

    def iz(j, carry):
        pltpu.sync_copy(rows.at[0], acc.at[pl.ds(s * RPT + j * KA, KA)])
        return carry
    lax.fori_loop(0, RPT // KA, iz, 0)
    pltpu.sync_copy(rows.at[0, pl.ds(0, 15)],
                    acc.at[pl.ds(s * RPT + (RPT // KA) * KA, 15)])
    for b in range(NB):
        pltpu.async_copy(u_hbm.at[sidx2.at[b]], rows.at[b], gsem[b])
    plsc.subcore_barrier()

    def round_(j, carry):
        for b in range(NB):
            i = j * NB + b
            pltpu.make_async_copy(
                u_hbm.at[sidx2.at[i]], rows.at[b], gsem[b]).wait()
            pltpu.async_copy(rows.at[b], acc.at[didx2.at[i]], ssem[b],
                             add=True)
        for b in range(NB):
            i = j * NB + b
            pltpu.make_async_copy(
                rows.at[b], acc.at[didx2.at[i]], ssem[b]).wait()

            @pl.when(j < CA // NB - 1)
            def _():
                pltpu.async_copy(
                    u_hbm.at[sidx2.at[i + NB]], rows.at[b], gsem[b])
        return carry
    lax.fori_loop(0, CA // NB, round_, 0)

    plsc.subcore_barrier()
    def ro(j, carry):
        pltpu.sync_copy(acc.at[pl.ds(s * RPT + j * KA, KA)], rows.at[0])
        pltpu.sync_copy(rows.at[0],
                        out_hbm.at[pl.ds(c * NPAD + s * RPT + j * KA, KA)])
        return carry
    lax.fori_loop(0, RPT // KA, ro, 0)
    rem = RPT - (RPT // KA) * KA
    pltpu.sync_copy(acc.at[pl.ds(s * RPT + (RPT // KA) * KA, rem)],
                    rows.at[1, pl.ds(0, rem)])
    pltpu.sync_copy(rows.at[1, pl.ds(0, rem)],
                    out_hbm.at[pl.ds(c * NPAD + s * RPT + (RPT // KA) * KA,
                                     rem)])


_R = 1024
_GRID = NPAD // _R
_A1 = NPAD // _R


def _u1_body(x_ref, w_ref, dinv_ref, o_ref):
    o_ref[...] = dinv_ref[...] * jnp.dot(
        x_ref[...], w_ref[...], preferred_element_type=jnp.float32)


def _u1(x, W1, dinv2):
    return pl.pallas_call(
        _u1_body,
        grid=(_GRID,),
        in_specs=[
            pl.BlockSpec((_R, DIN), lambda i: (i, 0)),
            pl.BlockSpec((DIN, DH), lambda i: (0, 0)),
            pl.BlockSpec((_R, 1), lambda i: (i, 0)),
        ],
        out_specs=pl.BlockSpec((_R, DH), lambda i: (i, 0)),
        out_shape=jax.ShapeDtypeStruct((N, DH), jnp.float32),
    )(x, W1, dinv2)


_R2 = _R // 2
_A1P = (NPAD // 2) // _R2
DH2 = 2 * DH


def _dfull(dinv_ref):
    d0 = jnp.broadcast_to(dinv_ref[:, 0:1], (dinv_ref.shape[0], DH))
    d1 = jnp.broadcast_to(dinv_ref[:, 1:2], (dinv_ref.shape[0], DH))
    return jnp.concatenate([d0, d1], axis=1)


def _trans_body(a0_ref, a1_ref, u_ref, dinv_ref, b_ref, w_ref, o_ref):
    d = _dfull(dinv_ref)
    h = jnp.maximum(
        d * (a0_ref[...] + a1_ref[...] + u_ref[...]) + b_ref[...], 0.0)
    o_ref[...] = d * jnp.dot(h, w_ref[...],
                             preferred_element_type=jnp.float32)


def _trans(aggp, up, dinvp, b_dup, Wbd):
    return pl.pallas_call(
        _trans_body,
        grid=(_GRID,),
        in_specs=[
            pl.BlockSpec((_R2, DH2), lambda i: (i, 0)),
            pl.BlockSpec((_R2, DH2), lambda i: (_A1P + i, 0)),
            pl.BlockSpec((_R2, DH2), lambda i: (i, 0)),
            pl.BlockSpec((_R2, 2), lambda i: (i, 0)),
            pl.BlockSpec((1, DH2), lambda i: (0, 0)),
            pl.BlockSpec((DH2, DH2), lambda i: (0, 0)),
        ],
        out_specs=pl.BlockSpec((_R2, DH2), lambda i: (i, 0)),
        out_shape=jax.ShapeDtypeStruct((N // 2, DH2), jnp.float32),
    )(aggp, aggp, up, dinvp, b_dup, Wbd)


def _final_body(a0_ref, a1_ref, u_ref, dinv_ref, b_ref, bf_ref, wl_ref,
                bl_ref, o_ref, sums, cnts):
    step = pl.program_id(0)

    @pl.when(step == 0)
    def _():
        sums[...] = jnp.zeros_like(sums)
        cnts[...] = jnp.zeros_like(cnts)

    pid = lax.broadcasted_iota(jnp.int32, (_R2, 1), 0) + step * _R2
    valid = pid < N // 2
    d = _dfull(dinv_ref)
    h = d * (a0_ref[...] + a1_ref[...] + u_ref[...]) + b_ref[...]
    h = jnp.where(valid, h, 0.0)
    gid = lax.broadcasted_iota(jnp.int32, (1, G), 1).astype(jnp.float32)
    vf = valid.astype(jnp.float32)
    oe = (bf_ref[:, 0:1] == gid).astype(jnp.float32) * vf
    oo = (bf_ref[:, 1:2] == gid).astype(jnp.float32) * vf
    dn = (((0,), (0,)), ((), ()))
    sums[...] += (
        lax.dot_general(oe, h[:, :DH], dn, preferred_element_type=jnp.float32)
        + lax.dot_general(oo, h[:, DH:], dn,
                          preferred_element_type=jnp.float32))
    ones_col = jnp.ones((_R2, 1), jnp.float32)
    cnts[...] += (
        lax.dot_general(oe, ones_col, dn, preferred_element_type=jnp.float32)
        + lax.dot_general(oo, ones_col, dn,
                          preferred_element_type=jnp.float32))

    @pl.when(step == pl.num_programs(0) - 1)
    def _():
        g = sums[...] / jnp.maximum(cnts[...], 1.0)
        o_ref[...] = jnp.dot(g, wl_ref[...],
                             preferred_element_type=jnp.float32) + bl_ref[...]


def _final(aggp, up, dinvp, b_dup, batchp, Wl, bl2):
    return pl.pallas_call(
        _final_body,
        grid=(_GRID,),
        in_specs=[
            pl.BlockSpec((_R2, DH2), lambda i: (i, 0)),
            pl.BlockSpec((_R2, DH2), lambda i: (_A1P + i, 0)),
            pl.BlockSpec((_R2, DH2), lambda i: (i, 0)),
            pl.BlockSpec((_R2, 2), lambda i: (i, 0)),
            pl.BlockSpec((1, DH2), lambda i: (0, 0)),
            pl.BlockSpec((_R2, 2), lambda i: (i, 0)),
            pl.BlockSpec((DH, 1), lambda i: (0, 0)),
            pl.BlockSpec((1, 1), lambda i: (0, 0)),
        ],
        out_specs=pl.BlockSpec((G, 1), lambda i: (0, 0)),
        out_shape=jax.ShapeDtypeStruct((G, 1), jnp.float32),
        scratch_shapes=[
            pltpu.VMEM((G, DH), jnp.float32),
            pltpu.VMEM((G, 1), jnp.float32),
        ],
    )(aggp, aggp, up, dinvp, b_dup, batchp, Wl, bl2)


def _blockdiag(W):
    Z = jnp.zeros((DH, DH), jnp.float32)
    return jnp.concatenate([jnp.concatenate([W, Z], 1),
                            jnp.concatenate([Z, W], 1)], 0)


def kernel(x, edge_index, batch, W1, b1, W2, b2, W3, b3, Wl, bl):
    src_a = edge_index[0].astype(jnp.int32).reshape(NW, CA, KA)
    dst_a = edge_index[1].astype(jnp.int32).reshape(NW, CA, KA)
    batchp = batch.astype(jnp.float32).reshape(N // 2, 2)

    degp = _deg(dst_a)
    deg = degp.reshape(NC, NPAD).sum(0)[:N] + 1.0
    dinv = lax.rsqrt(deg)
    dinv2 = dinv[:, None]
    dinvp = dinv.reshape(N // 2, 2)

    b1d = jnp.concatenate([b1, b1])[None, :]
    b2d = jnp.concatenate([b2, b2])[None, :]
    b3d = jnp.concatenate([b3, b3])[None, :]

    u1 = _u1(x, W1, dinv2)
    a = _agg(u1, src_a, dst_a)
    u2p = _trans(a.reshape(NC * NPAD // 2, DH2), u1.reshape(N // 2, DH2),
                 dinvp, b1d, _blockdiag(W2))
    a = _agg(u2p.reshape(N, DH), src_a, dst_a)
    u3p = _trans(a.reshape(NC * NPAD // 2, DH2), u2p, dinvp, b2d,
                 _blockdiag(W3))
    a = _agg(u3p.reshape(N, DH), src_a, dst_a)
    return _final(a.reshape(NC * NPAD // 2, DH2), u3p, dinvp, b3d, batchp,
                  Wl, bl[None, :])


# speedup vs baseline: 46.9062x; 1.0126x over previous
import functools

import jax
import jax.numpy as jnp
from jax import lax
from jax.experimental import pallas as pl
from jax.experimental.pallas import tpu as pltpu
from jax.experimental.pallas import tpu_sc as plsc

N = 10000
NPAD = 10240
E = 320000
DIN = 128
DH = 64
G = 64
NC = 2
NS = 16
NW = NC * NS
EPW = E // NW
KA = 125
CA = EPW // KA
NB = 8
RPT = NPAD // NS

_MESH = dict(core_axis_name="c", subcore_axis_name="s")


@functools.partial(
    pl.kernel,
    out_type=jax.ShapeDtypeStruct((NC * NPAD,), jnp.float32),
    mesh=plsc.VectorSubcoreMesh(**_MESH),
    scratch_types=[
        pltpu.VMEM((CA, KA), jnp.int32),
        pltpu.VMEM((128,), jnp.float32),
        pltpu.VMEM((RPT,), jnp.float32),
        pltpu.VMEM_SHARED((NPAD,), jnp.float32),
        pltpu.SemaphoreType.DMA,
    ],
)
def _deg(dst_hbm, out_hbm, didx2, ones_v, buf_v, acc, sem):
    c = lax.axis_index("c")
    s = lax.axis_index("s")
    wid = s * NC + c

    def fill_ones(i, carry):
        ones_v[pl.ds(i * 16, 16)] = jnp.ones((16,), jnp.float32)
        return carry
    lax.fori_loop(0, 128 // 16, fill_ones, 0)

    def fill_zeros(i, carry):
        buf_v[pl.ds(i * 16, 16)] = jnp.zeros((16,), jnp.float32)
        return carry
    lax.fori_loop(0, RPT // 16, fill_zeros, 0)

    pltpu.sync_copy(buf_v, acc.at[pl.ds(s * RPT, RPT)])
    pltpu.sync_copy(dst_hbm.at[wid], didx2)
    plsc.subcore_barrier()

    def chunk(i, carry):
        pltpu.async_copy(ones_v.at[pl.ds(0, KA)], acc.at[didx2.at[i]], sem,
                         add=True)

        @pl.when(i >= 4)
        def _():
            pltpu.make_async_copy(
                ones_v.at[pl.ds(0, KA)], acc.at[didx2.at[0]], sem).wait()
        return carry
    lax.fori_loop(0, CA, chunk, 0)
    for _ in range(4):
        pltpu.make_async_copy(
            ones_v.at[pl.ds(0, KA)], acc.at[didx2.at[0]], sem).wait()

    plsc.subcore_barrier()
    pltpu.sync_copy(acc.at[pl.ds(s * RPT, RPT)], buf_v)
    pltpu.sync_copy(buf_v, out_hbm.at[pl.ds(c * NPAD + s * RPT, RPT)])


@functools.partial(
    pl.kernel,
    out_type=jax.ShapeDtypeStruct((NC * NPAD, DH), jnp.float32),
    mesh=plsc.VectorSubcoreMesh(**_MESH),
    scratch_types=[
        pltpu.VMEM((CA, KA), jnp.int32),
        pltpu.VMEM((CA, KA), jnp.int32),
        pltpu.VMEM((NB, KA, DH), jnp.float32),
        pltpu.VMEM_SHARED((NPAD, DH), jnp.float32),
        [pltpu.SemaphoreType.DMA] * NB,
        [pltpu.SemaphoreType.DMA] * NB,
    ],
    compiler_params=pltpu.CompilerParams(use_tc_tiling_on_sc=False),
)
def _agg(u_hbm, src_hbm, dst_hbm, out_hbm, sidx2, didx2, rows, acc,
         gsem, ssem):
    c = lax.axis_index("c")
    s = lax.axis_index("s")
    wid = s * NC + c

    def fz(i, carry):
        for j in range(DH // 16):
            rows[0, i, pl.ds(j * 16, 16)] = jnp.zeros((16,), jnp.float32)
        return carry
    lax.fori_loop(0, KA, fz, 0)

    pltpu.sync_copy(src_hbm.at[wid], sidx2)
    pltpu.sync_copy(dst_hbm.at[wid], didx2)

    for j in range(RPT // KA):
        pltpu.async_copy(rows.at[0], acc.at[pl.ds(s * RPT + j * KA, KA)],
                         ssem[j])
    pltpu.async_copy(rows.at[0, pl.ds(0, 15)],
                     acc.at[pl.ds(s * RPT + (RPT // KA) * KA, 15)],
                     ssem[RPT // KA])
    for j in range(RPT // KA):
        pltpu.make_async_copy(rows.at[0],
                              acc.at[pl.ds(s * RPT + j * KA, KA)],
                              ssem[j]).wait()
    pltpu.make_async_copy(rows.at[0, pl.ds(0, 15)],
                          acc.at[pl.ds(s * RPT + (RPT // KA) * KA, 15)],
                          ssem[RPT // KA]).wait()
    for b in range(NB):
        pltpu.async_copy(u_hbm.at[sidx2.at[b]], rows.at[b], gsem[b])
    plsc.subcore_barrier()

    def round_(j, carry):
        for b in range(NB):
            i = j * NB + b
            pltpu.make_async_copy(
                u_hbm.at[sidx2.at[i]], rows.at[b], gsem[b]).wait()
            pltpu.async_copy(rows.at[b], acc.at[didx2.at[i]], ssem[b],
                             add=True)
        for b in range(NB):
            i = j * NB + b
            pltpu.make_async_copy(
                rows.at[b], acc.at[didx2.at[i]], ssem[b]).wait()

            @pl.when(j < CA // NB - 1)
            def _():
                pltpu.async_copy(
                    u_hbm.at[sidx2.at[i + NB]], rows.at[b], gsem[b])
        return carry
    lax.fori_loop(0, CA // NB, round_, 0)

    plsc.subcore_barrier()
    def osl(j, length=KA):
        return out_hbm.at[pl.ds(c * NPAD + s * RPT + j * KA, length)]
    for j in range(RPT // KA):
        b = j % 2
        if j >= 2:
            pltpu.make_async_copy(rows.at[b], osl(j - 2), gsem[b]).wait()
        pltpu.sync_copy(acc.at[pl.ds(s * RPT + j * KA, KA)], rows.at[b])
        pltpu.async_copy(rows.at[b], osl(j), gsem[b])
    pltpu.make_async_copy(rows.at[1], osl(3), gsem[1]).wait()
    pltpu.make_async_copy(rows.at[0], osl(4), gsem[0]).wait()
    rem = RPT - (RPT // KA) * KA
    pltpu.sync_copy(acc.at[pl.ds(s * RPT + (RPT // KA) * KA, rem)],
                    rows.at[2, pl.ds(0, rem)])
    pltpu.sync_copy(rows.at[2, pl.ds(0, rem)],
                    osl(RPT // KA, rem))


_R = 1024
_GRID = NPAD // _R
_A1 = NPAD // _R


def _u1_body(x_ref, w_ref, dinv_ref, o_ref):
    o_ref[...] = dinv_ref[...] * jnp.dot(
        x_ref[...], w_ref[...], preferred_element_type=jnp.float32)


def _u1(x, W1, dinv2):
    return pl.pallas_call(
        _u1_body,
        grid=(_GRID,),
        in_specs=[
            pl.BlockSpec((_R, DIN), lambda i: (i, 0)),
            pl.BlockSpec((DIN, DH), lambda i: (0, 0)),
            pl.BlockSpec((_R, 1), lambda i: (i, 0)),
        ],
        out_specs=pl.BlockSpec((_R, DH), lambda i: (i, 0)),
        out_shape=jax.ShapeDtypeStruct((N, DH), jnp.float32),
    )(x, W1, dinv2)


_R2 = _R // 2
_A1P = (NPAD // 2) // _R2
DH2 = 2 * DH


def _dfull(dinv_ref):
    d0 = jnp.broadcast_to(dinv_ref[:, 0:1], (dinv_ref.shape[0], DH))
    d1 = jnp.broadcast_to(dinv_ref[:, 1:2], (dinv_ref.shape[0], DH))
    return jnp.concatenate([d0, d1], axis=1)


def _trans_body(a0_ref, a1_ref, u_ref, dinv_ref, b_ref, w_ref, o_ref):
    d = _dfull(dinv_ref)
    h = jnp.maximum(
        d * (a0_ref[...] + a1_ref[...] + u_ref[...]) + b_ref[...], 0.0)
    o_ref[...] = d * jnp.dot(h, w_ref[...],
                             preferred_element_type=jnp.float32)


def _trans(aggp, up, dinvp, b_dup, Wbd):
    return pl.pallas_call(
        _trans_body,
        grid=(_GRID,),
        in_specs=[
            pl.BlockSpec((_R2, DH2), lambda i: (i, 0)),
            pl.BlockSpec((_R2, DH2), lambda i: (_A1P + i, 0)),
            pl.BlockSpec((_R2, DH2), lambda i: (i, 0)),
            pl.BlockSpec((_R2, 2), lambda i: (i, 0)),
            pl.BlockSpec((1, DH2), lambda i: (0, 0)),
            pl.BlockSpec((DH2, DH2), lambda i: (0, 0)),
        ],
        out_specs=pl.BlockSpec((_R2, DH2), lambda i: (i, 0)),
        out_shape=jax.ShapeDtypeStruct((N // 2, DH2), jnp.float32),
    )(aggp, aggp, up, dinvp, b_dup, Wbd)


def _final_body(a0_ref, a1_ref, u_ref, dinv_ref, b_ref, bf_ref, wl_ref,
                bl_ref, o_ref, sums, cnts):
    step = pl.program_id(0)

    @pl.when(step == 0)
    def _():
        sums[...] = jnp.zeros_like(sums)
        cnts[...] = jnp.zeros_like(cnts)

    pid = lax.broadcasted_iota(jnp.int32, (_R2, 1), 0) + step * _R2
    valid = pid < N // 2
    d = _dfull(dinv_ref)
    h = d * (a0_ref[...] + a1_ref[...] + u_ref[...]) + b_ref[...]
    h = jnp.where(valid, h, 0.0)
    gid = lax.broadcasted_iota(jnp.int32, (1, G), 1).astype(jnp.float32)
    vf = valid.astype(jnp.float32)
    oe = (bf_ref[:, 0:1] == gid).astype(jnp.float32) * vf
    oo = (bf_ref[:, 1:2] == gid).astype(jnp.float32) * vf
    dn = (((0,), (0,)), ((), ()))
    sums[...] += (
        lax.dot_general(oe, h[:, :DH], dn, preferred_element_type=jnp.float32)
        + lax.dot_general(oo, h[:, DH:], dn,
                          preferred_element_type=jnp.float32))
    ones_col = jnp.ones((_R2, 1), jnp.float32)
    cnts[...] += (
        lax.dot_general(oe, ones_col, dn, preferred_element_type=jnp.float32)
        + lax.dot_general(oo, ones_col, dn,
                          preferred_element_type=jnp.float32))

    @pl.when(step == pl.num_programs(0) - 1)
    def _():
        g = sums[...] / jnp.maximum(cnts[...], 1.0)
        o_ref[...] = jnp.dot(g, wl_ref[...],
                             preferred_element_type=jnp.float32) + bl_ref[...]


def _final(aggp, up, dinvp, b_dup, batchp, Wl, bl2):
    return pl.pallas_call(
        _final_body,
        grid=(_GRID,),
        in_specs=[
            pl.BlockSpec((_R2, DH2), lambda i: (i, 0)),
            pl.BlockSpec((_R2, DH2), lambda i: (_A1P + i, 0)),
            pl.BlockSpec((_R2, DH2), lambda i: (i, 0)),
            pl.BlockSpec((_R2, 2), lambda i: (i, 0)),
            pl.BlockSpec((1, DH2), lambda i: (0, 0)),
            pl.BlockSpec((_R2, 2), lambda i: (i, 0)),
            pl.BlockSpec((DH, 1), lambda i: (0, 0)),
            pl.BlockSpec((1, 1), lambda i: (0, 0)),
        ],
        out_specs=pl.BlockSpec((G, 1), lambda i: (0, 0)),
        out_shape=jax.ShapeDtypeStruct((G, 1), jnp.float32),
        scratch_shapes=[
            pltpu.VMEM((G, DH), jnp.float32),
            pltpu.VMEM((G, 1), jnp.float32),
        ],
    )(aggp, aggp, up, dinvp, b_dup, batchp, Wl, bl2)


def _blockdiag(W):
    Z = jnp.zeros((DH, DH), jnp.float32)
    return jnp.concatenate([jnp.concatenate([W, Z], 1),
                            jnp.concatenate([Z, W], 1)], 0)


def kernel(x, edge_index, batch, W1, b1, W2, b2, W3, b3, Wl, bl):
    src_a = edge_index[0].astype(jnp.int32).reshape(NW, CA, KA)
    dst_a = edge_index[1].astype(jnp.int32).reshape(NW, CA, KA)
    batchp = batch.astype(jnp.float32).reshape(N // 2, 2)

    degp = _deg(dst_a)
    deg = degp.reshape(NC, NPAD).sum(0)[:N] + 1.0
    dinv = lax.rsqrt(deg)
    dinvp = dinv.reshape(N // 2, 2)

    b1d = jnp.concatenate([b1, b1])[None, :]
    b2d = jnp.concatenate([b2, b2])[None, :]
    b3d = jnp.concatenate([b3, b3])[None, :]

    u1 = _u1(x, W1, dinv[:, None])
    a = _agg(u1, src_a, dst_a)
    u2p = _trans(a.reshape(NC * NPAD // 2, DH2), u1.reshape(N // 2, DH2),
                 dinvp, b1d, _blockdiag(W2))
    a = _agg(u2p.reshape(N, DH), src_a, dst_a)
    u3p = _trans(a.reshape(NC * NPAD // 2, DH2), u2p, dinvp, b2d,
                 _blockdiag(W3))
    a = _agg(u3p.reshape(N, DH), src_a, dst_a)
    return _final(a.reshape(NC * NPAD // 2, DH2), u3p, dinvp, b3d, batchp,
                  Wl, bl[None, :])


# speedup vs baseline: 48.2356x; 1.0283x over previous
import functools

import jax
import jax.numpy as jnp
from jax import lax
from jax.experimental import pallas as pl
from jax.experimental.pallas import tpu as pltpu
from jax.experimental.pallas import tpu_sc as plsc

N = 10000
NPAD = 10240
E = 320000
DIN = 128
DH = 64
G = 64
NC = 2
NS = 16
NW = NC * NS
EPW = E // NW
KA = 125
CA = EPW // KA
NB = 8
RPT = NPAD // NS

_MESH = dict(core_axis_name="c", subcore_axis_name="s")


@functools.partial(
    pl.kernel,
    out_type=jax.ShapeDtypeStruct((NC * NPAD,), jnp.float32),
    mesh=plsc.VectorSubcoreMesh(**_MESH),
    scratch_types=[
        pltpu.VMEM((CA, KA), jnp.int32),
        pltpu.VMEM((128,), jnp.float32),
        pltpu.VMEM((RPT,), jnp.float32),
        pltpu.VMEM_SHARED((NPAD,), jnp.float32),
        pltpu.SemaphoreType.DMA,
    ],
)
def _deg(edge_hbm, out_hbm, didx2, ones_v, buf_v, acc, sem):
    c = lax.axis_index("c")
    s = lax.axis_index("s")
    wid = s * NC + c

    def fill_ones(i, carry):
        ones_v[pl.ds(i * 16, 16)] = jnp.ones((16,), jnp.float32)
        return carry
    lax.fori_loop(0, 128 // 16, fill_ones, 0)

    def fill_zeros(i, carry):
        buf_v[pl.ds(i * 16, 16)] = jnp.zeros((16,), jnp.float32)
        return carry
    lax.fori_loop(0, RPT // 16, fill_zeros, 0)

    pltpu.sync_copy(buf_v, acc.at[pl.ds(s * RPT, RPT)])
    pltpu.sync_copy(edge_hbm.at[1, wid], didx2)
    plsc.subcore_barrier()

    def chunk(i, carry):
        pltpu.async_copy(ones_v.at[pl.ds(0, KA)], acc.at[didx2.at[i]], sem,
                         add=True)

        @pl.when(i >= 4)
        def _():
            pltpu.make_async_copy(
                ones_v.at[pl.ds(0, KA)], acc.at[didx2.at[0]], sem).wait()
        return carry
    lax.fori_loop(0, CA, chunk, 0)
    for _ in range(4):
        pltpu.make_async_copy(
            ones_v.at[pl.ds(0, KA)], acc.at[didx2.at[0]], sem).wait()

    plsc.subcore_barrier()
    pltpu.sync_copy(acc.at[pl.ds(s * RPT, RPT)], buf_v)
    pltpu.sync_copy(buf_v, out_hbm.at[pl.ds(c * NPAD + s * RPT, RPT)])


@functools.partial(
    pl.kernel,
    out_type=jax.ShapeDtypeStruct((NC * NPAD, DH), jnp.float32),
    mesh=plsc.VectorSubcoreMesh(**_MESH),
    scratch_types=[
        pltpu.VMEM((CA, KA), jnp.int32),
        pltpu.VMEM((CA, KA), jnp.int32),
        pltpu.VMEM((NB, KA, DH), jnp.float32),
        pltpu.VMEM_SHARED((NPAD, DH), jnp.float32),
        [pltpu.SemaphoreType.DMA] * NB,
        [pltpu.SemaphoreType.DMA] * NB,
    ],
    compiler_params=pltpu.CompilerParams(use_tc_tiling_on_sc=False),
)
def _agg(u_hbm, edge_hbm, out_hbm, sidx2, didx2, rows, acc,
         gsem, ssem):
    c = lax.axis_index("c")
    s = lax.axis_index("s")
    wid = s * NC + c

    def fz(i, carry):
        for j in range(DH // 16):
            rows[0, i, pl.ds(j * 16, 16)] = jnp.zeros((16,), jnp.float32)
        return carry
    lax.fori_loop(0, KA, fz, 0)

    pltpu.sync_copy(edge_hbm.at[0, wid], sidx2)
    pltpu.sync_copy(edge_hbm.at[1, wid], didx2)

    for j in range(RPT // KA):
        pltpu.async_copy(rows.at[0], acc.at[pl.ds(s * RPT + j * KA, KA)],
                         ssem[j])
    pltpu.async_copy(rows.at[0, pl.ds(0, 15)],
                     acc.at[pl.ds(s * RPT + (RPT // KA) * KA, 15)],
                     ssem[RPT // KA])
    for j in range(RPT // KA):
        pltpu.make_async_copy(rows.at[0],
                              acc.at[pl.ds(s * RPT + j * KA, KA)],
                              ssem[j]).wait()
    pltpu.make_async_copy(rows.at[0, pl.ds(0, 15)],
                          acc.at[pl.ds(s * RPT + (RPT // KA) * KA, 15)],
                          ssem[RPT // KA]).wait()
    for b in range(NB):
        pltpu.async_copy(u_hbm.at[sidx2.at[b]], rows.at[b], gsem[b])
    plsc.subcore_barrier()

    def round_(j, carry):
        for b in range(NB):
            i = j * NB + b
            pltpu.make_async_copy(
                u_hbm.at[sidx2.at[i]], rows.at[b], gsem[b]).wait()
            pltpu.async_copy(rows.at[b], acc.at[didx2.at[i]], ssem[b],
                             add=True)
        for b in range(NB):
            i = j * NB + b
            pltpu.make_async_copy(
                rows.at[b], acc.at[didx2.at[i]], ssem[b]).wait()

            @pl.when(j < CA // NB - 1)
            def _():
                pltpu.async_copy(
                    u_hbm.at[sidx2.at[i + NB]], rows.at[b], gsem[b])
        return carry
    lax.fori_loop(0, CA // NB, round_, 0)

    plsc.subcore_barrier()
    def osl(j, length=KA):
        return out_hbm.at[pl.ds(c * NPAD + s * RPT + j * KA, length)]
    for j in range(RPT // KA):
        b = j % 2
        if j >= 2:
            pltpu.make_async_copy(rows.at[b], osl(j - 2), gsem[b]).wait()
        pltpu.sync_copy(acc.at[pl.ds(s * RPT + j * KA, KA)], rows.at[b])
        pltpu.async_copy(rows.at[b], osl(j), gsem[b])
    pltpu.make_async_copy(rows.at[1], osl(3), gsem[1]).wait()
    pltpu.make_async_copy(rows.at[0], osl(4), gsem[0]).wait()
    rem = RPT - (RPT // KA) * KA
    pltpu.sync_copy(acc.at[pl.ds(s * RPT + (RPT // KA) * KA, rem)],
                    rows.at[2, pl.ds(0, rem)])
    pltpu.sync_copy(rows.at[2, pl.ds(0, rem)],
                    osl(RPT // KA, rem))


_R = 1024
_GRID = NPAD // _R
_A1 = NPAD // _R


def _u1_body(x_ref, w_ref, dinv_ref, o_ref):
    o_ref[...] = dinv_ref[...] * jnp.dot(
        x_ref[...], w_ref[...], preferred_element_type=jnp.float32)


def _u1(x, W1, dinv2):
    return pl.pallas_call(
        _u1_body,
        grid=(_GRID,),
        in_specs=[
            pl.BlockSpec((_R, DIN), lambda i: (i, 0)),
            pl.BlockSpec((DIN, DH), lambda i: (0, 0)),
            pl.BlockSpec((_R, 1), lambda i: (i, 0)),
        ],
        out_specs=pl.BlockSpec((_R, DH), lambda i: (i, 0)),
        out_shape=jax.ShapeDtypeStruct((N, DH), jnp.float32),
    )(x, W1, dinv2)


_R2 = _R // 2
_A1P = (NPAD // 2) // _R2
DH2 = 2 * DH


def _dfull(dinv_ref):
    d0 = jnp.broadcast_to(dinv_ref[:, 0:1], (dinv_ref.shape[0], DH))
    d1 = jnp.broadcast_to(dinv_ref[:, 1:2], (dinv_ref.shape[0], DH))
    return jnp.concatenate([d0, d1], axis=1)


def _trans_body(a0_ref, a1_ref, u_ref, dinv_ref, b_ref, w_ref, o_ref):
    d = _dfull(dinv_ref)
    h = jnp.maximum(
        d * (a0_ref[...] + a1_ref[...] + u_ref[...]) + b_ref[...], 0.0)
    o_ref[...] = d * jnp.dot(h, w_ref[...],
                             preferred_element_type=jnp.float32)


def _trans(aggp, up, dinvp, b_dup, Wbd):
    return pl.pallas_call(
        _trans_body,
        grid=(_GRID,),
        in_specs=[
            pl.BlockSpec((_R2, DH2), lambda i: (i, 0)),
            pl.BlockSpec((_R2, DH2), lambda i: (_A1P + i, 0)),
            pl.BlockSpec((_R2, DH2), lambda i: (i, 0)),
            pl.BlockSpec((_R2, 2), lambda i: (i, 0)),
            pl.BlockSpec((1, DH2), lambda i: (0, 0)),
            pl.BlockSpec((DH2, DH2), lambda i: (0, 0)),
        ],
        out_specs=pl.BlockSpec((_R2, DH2), lambda i: (i, 0)),
        out_shape=jax.ShapeDtypeStruct((N // 2, DH2), jnp.float32),
    )(aggp, aggp, up, dinvp, b_dup, Wbd)


def _final_body(a0_ref, a1_ref, u_ref, dinv_ref, b_ref, bf_ref, wl_ref,
                bl_ref, o_ref, sums, cnts):
    step = pl.program_id(0)

    @pl.when(step == 0)
    def _():
        sums[...] = jnp.zeros_like(sums)
        cnts[...] = jnp.zeros_like(cnts)

    pid = lax.broadcasted_iota(jnp.int32, (_R2, 1), 0) + step * _R2
    valid = pid < N // 2
    d = _dfull(dinv_ref)
    h = d * (a0_ref[...] + a1_ref[...] + u_ref[...]) + b_ref[...]
    h = jnp.where(valid, h, 0.0)
    gid = lax.broadcasted_iota(jnp.int32, (1, G), 1).astype(jnp.float32)
    vf = valid.astype(jnp.float32)
    oe = (bf_ref[:, 0:1] == gid).astype(jnp.float32) * vf
    oo = (bf_ref[:, 1:2] == gid).astype(jnp.float32) * vf
    dn = (((0,), (0,)), ((), ()))
    sums[...] += (
        lax.dot_general(oe, h[:, :DH], dn, preferred_element_type=jnp.float32)
        + lax.dot_general(oo, h[:, DH:], dn,
                          preferred_element_type=jnp.float32))
    ones_col = jnp.ones((_R2, 1), jnp.float32)
    cnts[...] += (
        lax.dot_general(oe, ones_col, dn, preferred_element_type=jnp.float32)
        + lax.dot_general(oo, ones_col, dn,
                          preferred_element_type=jnp.float32))

    @pl.when(step == pl.num_programs(0) - 1)
    def _():
        g = sums[...] / jnp.maximum(cnts[...], 1.0)
        o_ref[...] = jnp.dot(g, wl_ref[...],
                             preferred_element_type=jnp.float32) + bl_ref[...]


def _final(aggp, up, dinvp, b_dup, batchp, Wl, bl2):
    return pl.pallas_call(
        _final_body,
        grid=(_GRID,),
        in_specs=[
            pl.BlockSpec((_R2, DH2), lambda i: (i, 0)),
            pl.BlockSpec((_R2, DH2), lambda i: (_A1P + i, 0)),
            pl.BlockSpec((_R2, DH2), lambda i: (i, 0)),
            pl.BlockSpec((_R2, 2), lambda i: (i, 0)),
            pl.BlockSpec((1, DH2), lambda i: (0, 0)),
            pl.BlockSpec((_R2, 2), lambda i: (i, 0)),
            pl.BlockSpec((DH, 1), lambda i: (0, 0)),
            pl.BlockSpec((1, 1), lambda i: (0, 0)),
        ],
        out_specs=pl.BlockSpec((G, 1), lambda i: (0, 0)),
        out_shape=jax.ShapeDtypeStruct((G, 1), jnp.float32),
        scratch_shapes=[
            pltpu.VMEM((G, DH), jnp.float32),
            pltpu.VMEM((G, 1), jnp.float32),
        ],
    )(aggp, aggp, up, dinvp, b_dup, batchp, Wl, bl2)


def _blockdiag(W):
    Z = jnp.zeros((DH, DH), jnp.float32)
    return jnp.concatenate([jnp.concatenate([W, Z], 1),
                            jnp.concatenate([Z, W], 1)], 0)


def kernel(x, edge_index, batch, W1, b1, W2, b2, W3, b3, Wl, bl):
    edge3 = edge_index.astype(jnp.int32).reshape(2, NW, CA, KA)
    batchp = batch.astype(jnp.float32).reshape(N // 2, 2)

    degp = _deg(edge3)
    deg = degp.reshape(NC, NPAD).sum(0)[:N] + 1.0
    dinv = lax.rsqrt(deg)
    dinvp = dinv.reshape(N // 2, 2)

    b1d = jnp.concatenate([b1, b1])[None, :]
    b2d = jnp.concatenate([b2, b2])[None, :]
    b3d = jnp.concatenate([b3, b3])[None, :]

    u1 = _u1(x, W1, dinv[:, None])
    a = _agg(u1, edge3)
    u2p = _trans(a.reshape(NC * NPAD // 2, DH2), u1.reshape(N // 2, DH2),
                 dinvp, b1d, _blockdiag(W2))
    a = _agg(u2p.reshape(N, DH), edge3)
    u3p = _trans(a.reshape(NC * NPAD // 2, DH2), u2p, dinvp, b2d,
                 _blockdiag(W3))
    a = _agg(u3p.reshape(N, DH), edge3)
    return _final(a.reshape(NC * NPAD // 2, DH2), u3p, dinvp, b3d, batchp,
                  Wl, bl[None, :])
